# baseline (speedup 1.0000x reference)
import functools
import math

import jax
import jax.numpy as jnp
from jax import lax
from jax.experimental import pallas as pl

N_NODES = 10000
N_EDGES = 160000
F = 128
CUTOFF = 5.0
_LN2 = math.log(2.0)
_GAMMA = (127.0 / CUTOFF) ** 2
_STEP = CUTOFF / 127.0

BLK = 2000


def _ssp(v):
    return jnp.maximum(v, 0.0) + jnp.log(1.0 + jnp.exp(-jnp.abs(v))) - _LN2


def _edge_kernel(d_ref, ef_ref, g_ref, w1_ref, b1_ref, w2_ref, b2_ref, out_ref):
    d = d_ref[...]
    cut = 0.5 * (jnp.cos((math.pi / CUTOFF) * d) + 1.0)
    c = lax.broadcasted_iota(jnp.int32, (BLK, F), 1).astype(jnp.float32) * _STEP
    bf = jnp.exp(-_GAMMA * (d - c) ** 2)
    h1 = _ssp(jnp.dot(bf, w1_ref[...], preferred_element_type=jnp.float32) + b1_ref[...])
    h2 = _ssp(jnp.dot(h1, w2_ref[...], preferred_element_type=jnp.float32) + b2_ref[...])
    out_ref[...] = g_ref[...] * ef_ref[...] * h2 * cut


def kernel(x, edge_feat, dist, W1, b1, W2, b2, W3, b3, edge_index):
    src = edge_index[0]
    dst = edge_index[1]
    g = x[src]
    d2 = dist[:, None]
    grid = N_EDGES // BLK
    msg = pl.pallas_call(
        _edge_kernel,
        grid=(grid,),
        in_specs=[
            pl.BlockSpec((BLK, 1), lambda i: (i, 0)),
            pl.BlockSpec((BLK, F), lambda i: (i, 0)),
            pl.BlockSpec((BLK, F), lambda i: (i, 0)),
            pl.BlockSpec((F, F), lambda i: (0, 0)),
            pl.BlockSpec((1, F), lambda i: (0, 0)),
            pl.BlockSpec((F, F), lambda i: (0, 0)),
            pl.BlockSpec((1, F), lambda i: (0, 0)),
        ],
        out_specs=pl.BlockSpec((BLK, F), lambda i: (i, 0)),
        out_shape=jax.ShapeDtypeStruct((N_EDGES, F), jnp.float32),
    )(d2, edge_feat, g, W1, b1[None, :], W2, b2[None, :])
    h = jax.ops.segment_prod(msg, dst, num_segments=N_NODES)
    out = _ssp(h @ W3 + b3)
    return out


# speedup vs baseline: 1.9789x; 1.9789x over previous
import functools
import math

import jax
import jax.numpy as jnp
from jax import lax
from jax.experimental import pallas as pl
from jax.experimental.pallas import tpu as pltpu
from jax.experimental.pallas import tpu_sc as plsc

N_NODES = 10000
N_EDGES = 160000
F = 128
CUTOFF = 5.0
_LN2 = math.log(2.0)
_INV_LN2 = 1.0 / _LN2
_GAMMA = (127.0 / CUTOFF) ** 2
_STEP = CUTOFF / 127.0

EBLK = 2000
NBLK = 2000

NC = 2
NS = 16
NW = NC * NS
K = 128
ROWS = N_EDGES // K
ROWS_PAD = 1280
RPW = ROWS_PAD // NW
N_PAD = 10240
DUMP = N_PAD // NS


def _ssp(v):
    return jnp.maximum(v, 0.0) + jnp.log(1.0 + jnp.exp(-jnp.abs(v))) - _LN2


def _edge_kernel(d_ref, ef_ref, w1_ref, b1_ref, w2_ref, b2_ref, lp_ref, sp_ref):
    d = d_ref[...]
    cut = 0.5 * (jnp.cos((math.pi / CUTOFF) * d) + 1.0)
    c = lax.broadcasted_iota(jnp.int32, (EBLK, F), 1).astype(jnp.float32) * _STEP
    bf = jnp.exp(-_GAMMA * (d - c) ** 2)
    h1 = _ssp(jnp.dot(bf, w1_ref[...], preferred_element_type=jnp.float32) + b1_ref[...])
    h2 = _ssp(jnp.dot(h1, w2_ref[...], preferred_element_type=jnp.float32) + b2_ref[...])
    p = ef_ref[...] * h2 * cut
    lp_ref[...] = jnp.log(jnp.abs(p)) * _INV_LN2
    sp_ref[...] = jnp.where(p < 0.0, 1.0, 0.0)


def _node_kernel(x_ref, lx_ref, sx_ref):
    xv = x_ref[...]
    lx_ref[...] = jnp.log(jnp.abs(xv)) * _INV_LN2
    sx_ref[...] = jnp.where(xv < 0.0, 1.0, 0.0)


def _final_kernel(hl_ref, hn_ref, w3_ref, b3_ref, out_ref):
    hl = hl_ref[0] + hl_ref[1]
    n = hn_ref[0] + hn_ref[1]
    parity = n - 2.0 * jnp.floor(n * 0.5)
    sign = 1.0 - 2.0 * parity
    h = sign * jnp.exp(hl * _LN2)
    out_ref[...] = _ssp(jnp.dot(h, w3_ref[...], preferred_element_type=jnp.float32) + b3_ref[...])


def _sc_body(src_hbm, dst_hbm, lx_hbm, sx_hbm, lp_hbm, sp_hbm, zeros_hbm,
             hlog_hbm, hneg_hbm, sidx, didx, gbuf, pbuf, acc, sem):
    c = lax.axis_index("c")
    s = lax.axis_index("s")
    wid = s * NC + c

    nrows = jnp.where(wid == NW - 1, ROWS - RPW * (NW - 1), RPW)

    pltpu.sync_copy(src_hbm.at[pl.ds(RPW * wid, RPW)], sidx)
    pltpu.sync_copy(dst_hbm.at[pl.ds(RPW * wid, RPW)], didx)

    def phase(table_hbm, edge_hbm, out_hbm):
        pltpu.sync_copy(zeros_hbm, acc.at[pl.ds(DUMP * s, DUMP)])
        plsc.subcore_barrier()

        def body(j, _):
            r = RPW * wid + j
            pltpu.async_copy(table_hbm.at[sidx.at[j]], gbuf, sem).wait()
            pltpu.sync_copy(edge_hbm.at[pl.ds(r * K, K)], pbuf)
            pltpu.sync_copy(gbuf, acc.at[didx.at[j]], add=True)
            pltpu.sync_copy(pbuf, acc.at[didx.at[j]], add=True)
            return 0

        lax.fori_loop(0, nrows, body, 0)
        plsc.subcore_barrier()
        pltpu.sync_copy(acc.at[pl.ds(DUMP * s, DUMP)],
                        out_hbm.at[c, pl.ds(DUMP * s, DUMP)])
        plsc.subcore_barrier()

    phase(lx_hbm, lp_hbm, hlog_hbm)
    phase(sx_hbm, sp_hbm, hneg_hbm)


def kernel(x, edge_feat, dist, W1, b1, W2, b2, W3, b3, edge_index):
    idx2d = edge_index.astype(jnp.int32).reshape(2, ROWS, K)
    idx2d = jnp.pad(idx2d, ((0, 0), (0, ROWS_PAD - ROWS), (0, 0)))
    src2d, dst2d = idx2d[0], idx2d[1]
    d2 = dist[:, None]

    lp, sp = pl.pallas_call(
        _edge_kernel,
        grid=(N_EDGES // EBLK,),
        in_specs=[
            pl.BlockSpec((EBLK, 1), lambda i: (i, 0)),
            pl.BlockSpec((EBLK, F), lambda i: (i, 0)),
            pl.BlockSpec((F, F), lambda i: (0, 0)),
            pl.BlockSpec((1, F), lambda i: (0, 0)),
            pl.BlockSpec((F, F), lambda i: (0, 0)),
            pl.BlockSpec((1, F), lambda i: (0, 0)),
        ],
        out_specs=[
            pl.BlockSpec((EBLK, F), lambda i: (i, 0)),
            pl.BlockSpec((EBLK, F), lambda i: (i, 0)),
        ],
        out_shape=[
            jax.ShapeDtypeStruct((N_EDGES, F), jnp.float32),
            jax.ShapeDtypeStruct((N_EDGES, F), jnp.float32),
        ],
    )(d2, edge_feat, W1, b1[None, :], W2, b2[None, :])

    lx, sx = pl.pallas_call(
        _node_kernel,
        grid=(N_NODES // NBLK,),
        in_specs=[pl.BlockSpec((NBLK, F), lambda i: (i, 0))],
        out_specs=[
            pl.BlockSpec((NBLK, F), lambda i: (i, 0)),
            pl.BlockSpec((NBLK, F), lambda i: (i, 0)),
        ],
        out_shape=[
            jax.ShapeDtypeStruct((N_NODES, F), jnp.float32),
            jax.ShapeDtypeStruct((N_NODES, F), jnp.float32),
        ],
    )(x)

    zeros = jnp.zeros((DUMP, F), jnp.float32)
    mesh = plsc.VectorSubcoreMesh(core_axis_name="c", subcore_axis_name="s")
    sc = pl.kernel(
        _sc_body,
        out_type=[
            jax.ShapeDtypeStruct((NC, N_PAD, F), jnp.float32),
            jax.ShapeDtypeStruct((NC, N_PAD, F), jnp.float32),
        ],
        mesh=mesh,
        scratch_types=[
            pltpu.VMEM((RPW, K), jnp.int32),
            pltpu.VMEM((RPW, K), jnp.int32),
            pltpu.VMEM((K, F), jnp.float32),
            pltpu.VMEM((K, F), jnp.float32),
            pltpu.VMEM_SHARED((N_PAD, F), jnp.float32),
            pltpu.SemaphoreType.DMA,
        ],
    )
    hlog, hneg = sc(src2d, dst2d, lx, sx, lp, sp, zeros)

    out = pl.pallas_call(
        _final_kernel,
        grid=(N_NODES // NBLK,),
        in_specs=[
            pl.BlockSpec((NC, NBLK, F), lambda i: (0, i, 0)),
            pl.BlockSpec((NC, NBLK, F), lambda i: (0, i, 0)),
            pl.BlockSpec((F, F), lambda i: (0, 0)),
            pl.BlockSpec((1, F), lambda i: (0, 0)),
        ],
        out_specs=pl.BlockSpec((NBLK, F), lambda i: (i, 0)),
        out_shape=jax.ShapeDtypeStruct((N_NODES, F), jnp.float32),
    )(hlog, hneg, W3, b3[None, :])
    return out
